# baseline (speedup 1.0000x reference)
import jax
import jax.numpy as jnp
from jax.experimental import pallas as pl
from jax.experimental.pallas import tpu as pltpu

_PATCH = 7
_STRIDE = 2
_ALPHA = 0.05
_N = 2025
_NPAD = 2048
_D = 147
_DPAD = 256
_BIG = 1.0e30
_NBLK = 4
_BLK = _NPAD // _NBLK


def _extract_patches(x):
    patches = jax.lax.conv_general_dilated_patches(
        x, filter_shape=(_PATCH, _PATCH), window_strides=(_STRIDE, _STRIDE),
        padding='VALID')
    b, d, hh, ww = patches.shape
    return patches.reshape(b, d, hh * ww).transpose(0, 2, 1)


def _loss_kernel(inp_ref, tgt_ref, out_ref, colmin_ref):
    b = pl.program_id(0)
    k = pl.program_id(1)
    nb = pl.num_programs(0)

    inp = inp_ref[0]
    tgt = tgt_ref[0]
    dot = jax.lax.dot_general(
        tgt, inp, (((1,), (1,)), ((), ())),
        preferred_element_type=jnp.float32)

    rowmax = jnp.max(dot, axis=1, keepdims=True)
    denom = rowmax * (-2.0 / _D) + _ALPHA
    s = (-2.0 / _D) / denom
    rowid = jax.lax.broadcasted_iota(jnp.int32, (_BLK, 1), 0) + k * _BLK
    m = jnp.where(rowid < _N, 0.0, _BIG)
    v = dot * s + m
    cmin = jnp.min(v, axis=0, keepdims=True)

    @pl.when(k == 0)
    def _start_batch():
        colmin_ref[...] = cmin

    @pl.when(k > 0)
    def _accum():
        colmin_ref[...] = jnp.minimum(colmin_ref[...], cmin)

    @pl.when(jnp.logical_and(b == 0, k == 0))
    def _init_out():
        out_ref[...] = jnp.zeros_like(out_ref)

    @pl.when(k == _NBLK - 1)
    def _finish_batch():
        colmask = jax.lax.broadcasted_iota(jnp.int32, (1, _NPAD), 1) < _N
        loss_b = jnp.sum(jnp.where(colmask, colmin_ref[...], 0.0),
                         axis=1, keepdims=True) * (1.0 / _N)
        out_ref[...] += loss_b / nb


def _augment(xp, yp):
    bsz = xp.shape[0]
    pad_n = _NPAD - _N
    pad_d = _DPAD - _D - 2
    n2x = 0.5 * jnp.sum(xp * xp, axis=2)
    n2y = 0.5 * jnp.sum(yp * yp, axis=2)
    ones = jnp.ones((bsz, _N, 1), xp.dtype)
    n2x_p = jnp.pad(n2x, ((0, 0), (0, pad_n)), constant_values=_BIG * 0.5)
    key = jnp.concatenate([
        jnp.pad(xp, ((0, 0), (0, pad_n), (0, 0))),
        n2x_p[..., None],
        jnp.pad(ones, ((0, 0), (0, pad_n), (0, 0))),
    ], axis=2)
    key = jnp.pad(key, ((0, 0), (0, 0), (0, pad_d)))
    qry = jnp.concatenate([
        jnp.pad(yp, ((0, 0), (0, pad_n), (0, 0))),
        jnp.pad(-ones, ((0, 0), (0, pad_n), (0, 0))),
        jnp.pad(-n2y[..., None], ((0, 0), (0, pad_n), (0, 0))),
    ], axis=2)
    qry = jnp.pad(qry, ((0, 0), (0, 0), (0, pad_d)))
    return key, qry


def kernel(x, y):
    xp = _extract_patches(x)
    yp = _extract_patches(y)
    bsz = xp.shape[0]
    key, qry = _augment(xp, yp)

    out = pl.pallas_call(
        _loss_kernel,
        grid=(bsz, _NBLK),
        in_specs=[
            pl.BlockSpec((1, _NPAD, _DPAD), lambda b, k: (b, 0, 0)),
            pl.BlockSpec((1, _BLK, _DPAD), lambda b, k: (b, k, 0)),
        ],
        out_specs=pl.BlockSpec((1, 1), lambda b, k: (0, 0)),
        out_shape=jax.ShapeDtypeStruct((1, 1), jnp.float32),
        scratch_shapes=[pltpu.VMEM((1, _NPAD), jnp.float32)],
    )(key, qry)
    return out[0, 0]


# speedup vs baseline: 1.0447x; 1.0447x over previous
import jax
import jax.numpy as jnp
from jax.experimental import pallas as pl
from jax.experimental.pallas import tpu as pltpu

_PATCH = 7
_STRIDE = 2
_ALPHA = 0.05
_N = 2025
_NPAD = 2048
_D = 147
_DPAD = 256
_BIG = 1.0e30
_NBLK = 4
_BLK = _NPAD // _NBLK


def _extract_patches(x):
    patches = jax.lax.conv_general_dilated_patches(
        x, filter_shape=(_PATCH, _PATCH), window_strides=(_STRIDE, _STRIDE),
        padding='VALID')
    b, d, hh, ww = patches.shape
    return patches.reshape(b, d, hh * ww).transpose(0, 2, 1)


def _loss_kernel(inp_ref, tgt_ref, out_ref):
    b = pl.program_id(0)
    nb = pl.num_programs(0)

    inp = inp_ref[0]
    tgt = tgt_ref[0]
    dot = jax.lax.dot_general(
        tgt, inp, (((1,), (1,)), ((), ())),
        preferred_element_type=jnp.float32)

    rowmax = jnp.max(dot, axis=1, keepdims=True)
    denom = rowmax * (-2.0 / _D) + _ALPHA
    s = (-2.0 / _D) / denom
    rowid = jax.lax.broadcasted_iota(jnp.int32, (_NPAD, 1), 0)
    m = jnp.where(rowid < _N, 0.0, _BIG)
    cmin = jnp.min(dot * s + m, axis=0, keepdims=True)

    colmask = jax.lax.broadcasted_iota(jnp.int32, (1, _NPAD), 1) < _N
    loss_b = jnp.sum(jnp.where(colmask, cmin, 0.0),
                     axis=1, keepdims=True) * (1.0 / _N)

    @pl.when(b == 0)
    def _init_out():
        out_ref[...] = jnp.zeros_like(out_ref)

    out_ref[...] += loss_b / nb


def _augment(xp, yp):
    bsz = xp.shape[0]
    pad_n = _NPAD - _N
    pad_d = _DPAD - _D - 2
    n2x = 0.5 * jnp.sum(xp * xp, axis=2)
    n2y = 0.5 * jnp.sum(yp * yp, axis=2)
    ones = jnp.ones((bsz, _N, 1), xp.dtype)
    n2x_p = jnp.pad(n2x, ((0, 0), (0, pad_n)), constant_values=_BIG * 0.5)
    key = jnp.concatenate([
        jnp.pad(xp, ((0, 0), (0, pad_n), (0, 0))),
        n2x_p[..., None],
        jnp.pad(ones, ((0, 0), (0, pad_n), (0, 0))),
    ], axis=2)
    key = jnp.pad(key, ((0, 0), (0, 0), (0, pad_d)))
    qry = jnp.concatenate([
        jnp.pad(yp, ((0, 0), (0, pad_n), (0, 0))),
        jnp.pad(-ones, ((0, 0), (0, pad_n), (0, 0))),
        jnp.pad(-n2y[..., None], ((0, 0), (0, pad_n), (0, 0))),
    ], axis=2)
    qry = jnp.pad(qry, ((0, 0), (0, 0), (0, pad_d)))
    return key, qry


def kernel(x, y):
    xp = _extract_patches(x)
    yp = _extract_patches(y)
    bsz = xp.shape[0]
    key, qry = _augment(xp, yp)

    out = pl.pallas_call(
        _loss_kernel,
        grid=(bsz,),
        in_specs=[
            pl.BlockSpec((1, _NPAD, _DPAD), lambda b: (b, 0, 0)),
            pl.BlockSpec((1, _NPAD, _DPAD), lambda b: (b, 0, 0)),
        ],
        out_specs=pl.BlockSpec((1, 1), lambda b: (0, 0)),
        out_shape=jax.ShapeDtypeStruct((1, 1), jnp.float32),
    )(key, qry)
    return out[0, 0]


# speedup vs baseline: 1.2647x; 1.2105x over previous
import jax
import jax.numpy as jnp
from jax.experimental import pallas as pl
from jax.experimental.pallas import tpu as pltpu

_PATCH = 7
_STRIDE = 2
_ALPHA = 0.05
_N = 2025
_NPAD = 2048
_D = 147
_DPAD = 256
_BIG = 1.0e30


def _extract_patches(x):
    patches = jax.lax.conv_general_dilated_patches(
        x, filter_shape=(_PATCH, _PATCH), window_strides=(_STRIDE, _STRIDE),
        padding='VALID')
    b, d, hh, ww = patches.shape
    return patches.reshape(b, d, hh * ww).transpose(0, 2, 1)


def _loss_kernel(inp_ref, tgt_ref, out_ref):
    b = pl.program_id(0)
    nb = pl.num_programs(0)

    inp = inp_ref[0]
    tgt = tgt_ref[0]
    rowid = jax.lax.broadcasted_iota(jnp.int32, (_NPAD, 1), 0)

    xn = 0.5 * jnp.sum(inp * inp, axis=1, keepdims=True)
    yn = 0.5 * jnp.sum(tgt * tgt, axis=1, keepdims=True)
    inp_ref[0, :, _D:_D + 1] = jnp.where(rowid < _N, xn, _BIG * 0.5)
    inp_ref[0, :, _D + 1:_D + 2] = jnp.ones((_NPAD, 1), jnp.float32)
    tgt_ref[0, :, _D:_D + 1] = jnp.full((_NPAD, 1), -1.0, jnp.float32)
    tgt_ref[0, :, _D + 1:_D + 2] = -yn

    dot = jax.lax.dot_general(
        tgt_ref[0], inp_ref[0], (((1,), (1,)), ((), ())),
        preferred_element_type=jnp.float32)

    rowmax = jnp.max(dot, axis=1, keepdims=True)
    denom = rowmax * (-2.0 / _D) + _ALPHA
    s = (-2.0 / _D) / denom
    m = jnp.where(rowid < _N, 0.0, _BIG)
    cmin = jnp.min(dot * s + m, axis=0, keepdims=True)

    colmask = jax.lax.broadcasted_iota(jnp.int32, (1, _NPAD), 1) < _N
    loss_b = jnp.sum(jnp.where(colmask, cmin, 0.0),
                     axis=1, keepdims=True) * (1.0 / _N)

    @pl.when(b == 0)
    def _init_out():
        out_ref[...] = jnp.zeros_like(out_ref)

    out_ref[...] += loss_b / nb


def kernel(x, y):
    xp = _extract_patches(x)
    yp = _extract_patches(y)
    bsz = xp.shape[0]
    xp = jnp.pad(xp, ((0, 0), (0, _NPAD - _N), (0, _DPAD - _D)))
    yp = jnp.pad(yp, ((0, 0), (0, _NPAD - _N), (0, _DPAD - _D)))

    out = pl.pallas_call(
        _loss_kernel,
        grid=(bsz,),
        in_specs=[
            pl.BlockSpec((1, _NPAD, _DPAD), lambda b: (b, 0, 0)),
            pl.BlockSpec((1, _NPAD, _DPAD), lambda b: (b, 0, 0)),
        ],
        out_specs=pl.BlockSpec((1, 1), lambda b: (0, 0)),
        out_shape=jax.ShapeDtypeStruct((1, 1), jnp.float32),
    )(xp, yp)
    return out[0, 0]
